# Initial kernel scaffold; baseline (speedup 1.0000x reference)
import jax
import jax.numpy as jnp
from jax.experimental import pallas as pl


def kernel(inputs, codebook):
    raise NotImplementedError("write your pallas kernel here")


# speedup vs baseline: 5.6524x; 5.6524x over previous
import functools

import jax
import jax.numpy as jnp
from jax import lax
from jax.experimental import pallas as pl
from jax.experimental.pallas import tpu as pltpu
from jax.experimental.pallas import tpu_sc as plsc

VOCAB = 8192
DIM = 256
N_TOK = 4096
COMMIT = 0.25

TR = 1024
TV = 1024
RT = N_TOK // TR
VT = VOCAB // TV
TR_S = TR // 128
TV_S = TV // 128


def _argmin_body(x_ref, c_ref, minv_ref, mini_ref):
    v = pl.program_id(1)
    x = x_ref[...]
    c = c_ref[...]
    xsq = jnp.sum(x * x, axis=1, keepdims=True)
    csq = jnp.sum(c * c, axis=1)
    mm = lax.dot_general(x, c, (((1,), (1,)), ((), ())),
                         preferred_element_type=jnp.float32)
    d = (xsq + csq[None, :]) - 2.0 * mm
    lmin = jnp.min(d, axis=1, keepdims=True)
    cols = lax.broadcasted_iota(jnp.int32, d.shape, 1)
    larg = jnp.min(jnp.where(d == lmin, cols, VOCAB),
                   axis=1, keepdims=True) + v * TV
    lmin3 = lmin.reshape(1, TR, 1)
    larg3 = larg.reshape(1, TR, 1)

    @pl.when(v == 0)
    def _():
        minv_ref[...] = lmin3
        mini_ref[...] = larg3

    @pl.when(v > 0)
    def _():
        better = lmin3 < minv_ref[...]
        minv_ref[...] = jnp.where(better, lmin3, minv_ref[...])
        mini_ref[...] = jnp.where(better, larg3, mini_ref[...])


_argmin_call = pl.pallas_call(
    _argmin_body,
    grid=(RT, VT),
    in_specs=[
        pl.BlockSpec((TR, DIM), lambda r, v: (r, 0)),
        pl.BlockSpec((TV, DIM), lambda r, v: (v, 0)),
    ],
    out_specs=[
        pl.BlockSpec((1, TR, 1), lambda r, v: (r, 0, 0)),
        pl.BlockSpec((1, TR, 1), lambda r, v: (r, 0, 0)),
    ],
    out_shape=[
        jax.ShapeDtypeStruct((RT, TR, 1), jnp.float32),
        jax.ShapeDtypeStruct((RT, TR, 1), jnp.int32),
    ],
)


def _onehot_body(idx_ref, enc_ref, cnt_ref):
    v = pl.program_id(0)
    r = pl.program_id(1)
    idx = idx_ref[...].reshape(TR, 1)
    colg = lax.broadcasted_iota(jnp.int32, (TR, TV), 1) + v * TV
    onehot = (idx == colg).astype(jnp.float32)
    enc_ref[...] = onehot
    s = jnp.sum(onehot, axis=0, keepdims=True).reshape(1, 1, TV)

    @pl.when(r == 0)
    def _():
        cnt_ref[...] = s

    @pl.when(r > 0)
    def _():
        cnt_ref[...] = cnt_ref[...] + s


_onehot_call = pl.pallas_call(
    _onehot_body,
    grid=(VT, RT),
    in_specs=[
        pl.BlockSpec((1, TR, 1), lambda v, r: (r, 0, 0)),
    ],
    out_specs=[
        pl.BlockSpec((TR, TV), lambda v, r: (r, v)),
        pl.BlockSpec((1, 1, TV), lambda v, r: (v, 0, 0)),
    ],
    out_shape=[
        jax.ShapeDtypeStruct((N_TOK, VOCAB), jnp.float32),
        jax.ShapeDtypeStruct((VT, 1, TV), jnp.float32),
    ],
)


_NC = 2
_NS = 16
_NW = _NC * _NS
_BPW = N_TOK // _NW


@functools.cache
def _make_sc_gather():
    @functools.partial(
        pl.kernel,
        mesh=plsc.VectorSubcoreMesh(core_axis_name="c", subcore_axis_name="s"),
        out_type=jax.ShapeDtypeStruct((N_TOK, DIM), jnp.float32),
        scratch_types=[
            pltpu.VMEM((_BPW,), jnp.int32),
            pltpu.VMEM((_BPW, DIM), jnp.float32),
            pltpu.SemaphoreType.DMA,
        ],
    )
    def _sc_gather(table_hbm, idx_hbm, out_hbm, idx_v, rows_v, sem):
        wid = lax.axis_index("s") * _NC + lax.axis_index("c")
        base = wid * _BPW
        pltpu.sync_copy(idx_hbm.at[pl.ds(base, _BPW)], idx_v)
        pltpu.async_copy(table_hbm.at[idx_v], rows_v, sem).wait()
        pltpu.sync_copy(rows_v, out_hbm.at[pl.ds(base, _BPW)])

    return _sc_gather


def _finish_body(x_ref, q_ref, cnt_ref, qst_ref, loss_ref, perp_ref, acc_ref):
    r = pl.program_id(0)
    x = x_ref[...]
    q = q_ref[...]
    diff = q - x
    qst_ref[...] = x + diff
    s = jnp.sum(diff * diff)

    @pl.when(r == 0)
    def _():
        acc_ref[0, 0] = s

    @pl.when(r > 0)
    def _():
        acc_ref[0, 0] = acc_ref[0, 0] + s

    @pl.when(r == RT - 1)
    def _():
        m = acc_ref[0, 0] * (1.0 / (N_TOK * DIM))
        loss_ref[0, 0] = m + COMMIT * m
        p = cnt_ref[...].reshape(VT, TV) * (1.0 / N_TOK)
        perp_ref[0, 0] = jnp.exp(-jnp.sum(p * jnp.log(p + 1e-10)))


_finish_call = pl.pallas_call(
    _finish_body,
    grid=(RT,),
    in_specs=[
        pl.BlockSpec((TR, DIM), lambda r: (r, 0)),
        pl.BlockSpec((TR, DIM), lambda r: (r, 0)),
        pl.BlockSpec((VT, 1, TV), lambda r: (0, 0, 0)),
    ],
    out_specs=[
        pl.BlockSpec((TR, DIM), lambda r: (r, 0)),
        pl.BlockSpec(memory_space=pltpu.SMEM),
        pl.BlockSpec(memory_space=pltpu.SMEM),
    ],
    out_shape=[
        jax.ShapeDtypeStruct((N_TOK, DIM), jnp.float32),
        jax.ShapeDtypeStruct((1, 1), jnp.float32),
        jax.ShapeDtypeStruct((1, 1), jnp.float32),
    ],
    scratch_shapes=[pltpu.SMEM((1, 1), jnp.float32)],
)


def kernel(inputs, codebook):
    B, C, H, W = inputs.shape
    x = jnp.transpose(inputs, (0, 2, 3, 1)).reshape(N_TOK, DIM)
    _, mini = _argmin_call(x, codebook)
    enc, cnt = _onehot_call(mini)
    idx_flat = mini.reshape(N_TOK)
    q = _make_sc_gather()(codebook, idx_flat)
    qst, loss, perp = _finish_call(x, q, cnt)
    qst_out = jnp.transpose(qst.reshape(B, H, W, C), (0, 3, 1, 2))
    return (loss.reshape(()), qst_out, perp.reshape(()), enc)
